# Initial kernel scaffold; baseline (speedup 1.0000x reference)
import jax
import jax.numpy as jnp
from jax.experimental import pallas as pl

_DIM = 1024
_HALF = _DIM // 2
_L0 = 64
_L1 = 64
_BLK0 = 8


def _emb_kernel(w0_ref, b0_ref, w1_ref, b1_ref, scal_ref, out_ref):
    p = pl.program_id(0)
    div0 = scal_ref[0, 0]
    mult0 = scal_ref[0, 1]
    div1 = scal_ref[0, 2]
    mult1 = scal_ref[0, 3]

    c0 = jax.lax.broadcasted_iota(jnp.float32, (_BLK0, 1), 0) + (
        p * _BLK0).astype(jnp.float32)
    pos0 = c0 / div0 * mult0
    emb0 = jnp.sin(pos0 * w0_ref[...] + b0_ref[...])
    out_ref[:, :, :_HALF] = jnp.broadcast_to(
        emb0[:, None, :], (_BLK0, _L1, _HALF))

    c1 = jax.lax.broadcasted_iota(jnp.float32, (_L1, 1), 0)
    pos1 = c1 / div1 * mult1
    emb1 = jnp.sin(pos1 * w1_ref[...] + b1_ref[...])
    out_ref[:, :, _HALF:] = jnp.broadcast_to(
        emb1[None, :, :], (_BLK0, _L1, _HALF))


def kernel(seq_len_or_axial_dims, W0, b0, W1, b1, div0, mult0, div1, mult1):
    w0 = W0.reshape(1, _HALF)
    b0r = b0.reshape(1, _HALF)
    w1 = W1.reshape(1, _HALF)
    b1r = b1.reshape(1, _HALF)
    scal = jnp.stack([div0, mult0, div1, mult1]).reshape(1, 4)

    grid = (_L0 // _BLK0,)
    out = pl.pallas_call(
        _emb_kernel,
        grid=grid,
        in_specs=[
            pl.BlockSpec((1, _HALF), lambda p: (0, 0)),
            pl.BlockSpec((1, _HALF), lambda p: (0, 0)),
            pl.BlockSpec((1, _HALF), lambda p: (0, 0)),
            pl.BlockSpec((1, _HALF), lambda p: (0, 0)),
            pl.BlockSpec((1, 4), lambda p: (0, 0)),
        ],
        out_specs=pl.BlockSpec((_BLK0, _L1, _DIM), lambda p: (p, 0, 0)),
        out_shape=jax.ShapeDtypeStruct((_L0, _L1, _DIM), jnp.float32),
    )(w0, b0r, w1, b1r, scal)
    return out.reshape(_L0 * _L1, _DIM)


# speedup vs baseline: 4.1207x; 4.1207x over previous
import jax
import jax.numpy as jnp
from jax.experimental import pallas as pl

_DIM = 1024
_HALF = _DIM // 2
_L0 = 64
_L1 = 64
_BLK0 = 8


def _emb_kernel(w0_ref, b0_ref, w1_ref, b1_ref, scal_ref, out_ref):
    p = pl.program_id(0)
    div0 = scal_ref[0, 0]
    mult0 = scal_ref[0, 1]
    div1 = scal_ref[0, 2]
    mult1 = scal_ref[0, 3]

    c0 = (jax.lax.broadcasted_iota(jnp.int32, (_BLK0, 1), 0)
          + p * _BLK0).astype(jnp.float32)
    pos0 = c0 / div0 * mult0
    emb0 = jnp.sin(pos0 * w0_ref[...] + b0_ref[...])
    out_ref[:, :, :_HALF] = jnp.broadcast_to(
        emb0[:, None, :], (_BLK0, _L1, _HALF))

    c1 = jax.lax.broadcasted_iota(jnp.int32, (_L1, 1), 0).astype(jnp.float32)
    pos1 = c1 / div1 * mult1
    emb1 = jnp.sin(pos1 * w1_ref[...] + b1_ref[...])
    out_ref[:, :, _HALF:] = jnp.broadcast_to(
        emb1[None, :, :], (_BLK0, _L1, _HALF))


def kernel(seq_len_or_axial_dims, W0, b0, W1, b1, div0, mult0, div1, mult1):
    w0 = W0.reshape(1, _HALF)
    b0r = b0.reshape(1, _HALF)
    w1 = W1.reshape(1, _HALF)
    b1r = b1.reshape(1, _HALF)
    scal = jnp.stack([div0, mult0, div1, mult1]).reshape(1, 4)

    grid = (_L0 // _BLK0,)
    out = pl.pallas_call(
        _emb_kernel,
        grid=grid,
        in_specs=[
            pl.BlockSpec((1, _HALF), lambda p: (0, 0)),
            pl.BlockSpec((1, _HALF), lambda p: (0, 0)),
            pl.BlockSpec((1, _HALF), lambda p: (0, 0)),
            pl.BlockSpec((1, _HALF), lambda p: (0, 0)),
            pl.BlockSpec((1, 4), lambda p: (0, 0)),
        ],
        out_specs=pl.BlockSpec((_BLK0, _L1, _DIM), lambda p: (p, 0, 0)),
        out_shape=jax.ShapeDtypeStruct((_L0, _L1, _DIM), jnp.float32),
    )(w0, b0r, w1, b1r, scal)
    return out.reshape(_L0 * _L1, _DIM)


# speedup vs baseline: 4.4480x; 1.0794x over previous
import jax
import jax.numpy as jnp
from jax.experimental import pallas as pl
from jax.experimental.pallas import tpu as pltpu

_DIM = 1024
_HALF = _DIM // 2
_L0 = 64
_L1 = 64
_BLK0 = 8


def _emb_kernel(w0_ref, b0_ref, w1_ref, b1_ref, scal_ref, out_ref, emb1_ref):
    p = pl.program_id(0)

    @pl.when(p == 0)
    def _():
        div1 = scal_ref[0, 2]
        mult1 = scal_ref[0, 3]
        c1 = jax.lax.broadcasted_iota(jnp.int32, (_L1, 1), 0).astype(
            jnp.float32)
        pos1 = c1 / div1 * mult1
        emb1_ref[...] = jnp.sin(pos1 * w1_ref[...] + b1_ref[...])

    div0 = scal_ref[0, 0]
    mult0 = scal_ref[0, 1]
    c0 = (jax.lax.broadcasted_iota(jnp.int32, (_BLK0, 1), 0)
          + p * _BLK0).astype(jnp.float32)
    pos0 = c0 / div0 * mult0
    emb0 = jnp.sin(pos0 * w0_ref[...] + b0_ref[...])
    out_ref[:, :, :_HALF] = jnp.broadcast_to(
        emb0[:, None, :], (_BLK0, _L1, _HALF))
    out_ref[:, :, _HALF:] = jnp.broadcast_to(
        emb1_ref[...][None, :, :], (_BLK0, _L1, _HALF))


def kernel(seq_len_or_axial_dims, W0, b0, W1, b1, div0, mult0, div1, mult1):
    w0 = W0.reshape(1, _HALF)
    b0r = b0.reshape(1, _HALF)
    w1 = W1.reshape(1, _HALF)
    b1r = b1.reshape(1, _HALF)
    scal = jnp.stack([div0, mult0, div1, mult1]).reshape(1, 4)

    grid = (_L0 // _BLK0,)
    out = pl.pallas_call(
        _emb_kernel,
        grid=grid,
        in_specs=[
            pl.BlockSpec((1, _HALF), lambda p: (0, 0)),
            pl.BlockSpec((1, _HALF), lambda p: (0, 0)),
            pl.BlockSpec((1, _HALF), lambda p: (0, 0)),
            pl.BlockSpec((1, _HALF), lambda p: (0, 0)),
            pl.BlockSpec((1, 4), lambda p: (0, 0)),
        ],
        out_specs=pl.BlockSpec((_BLK0, _L1, _DIM), lambda p: (p, 0, 0)),
        out_shape=jax.ShapeDtypeStruct((_L0, _L1, _DIM), jnp.float32),
        scratch_shapes=[pltpu.VMEM((_L1, _HALF), jnp.float32)],
    )(w0, b0r, w1, b1r, scal)
    return out.reshape(_L0 * _L1, _DIM)
